# baseline (speedup 1.0000x reference)
import functools

import jax
import jax.numpy as jnp
from jax import lax
from jax.experimental import pallas as pl
from jax.experimental.pallas import tpu as pltpu
from jax.experimental.pallas import tpu_sc as plsc

N = 10000
E = 320000
D = 128

CHUNK = 80
NROWCH = E // CHUNK
TILES = 32
PER_TILE = NROWCH // TILES

ROW_BLK = 1000


def _mm_att_body(x_ref, w_ref, asrc_ref, adst_ref, h_ref, as_ref, ad_ref):
    h = jnp.dot(x_ref[...], w_ref[...], preferred_element_type=jnp.float32)
    h_ref[...] = h
    as_ref[...] = jnp.sum(h * asrc_ref[...], axis=-1, keepdims=True)
    ad_ref[...] = jnp.sum(h * adst_ref[...], axis=-1, keepdims=True)


def _tc_gat_mm(x, W_gat, att_src, att_dst):
    grid = (N // ROW_BLK,)
    return pl.pallas_call(
        _mm_att_body,
        grid=grid,
        in_specs=[
            pl.BlockSpec((ROW_BLK, D), lambda i: (i, 0)),
            pl.BlockSpec((D, D), lambda i: (0, 0)),
            pl.BlockSpec((1, D), lambda i: (0, 0)),
            pl.BlockSpec((1, D), lambda i: (0, 0)),
        ],
        out_specs=[
            pl.BlockSpec((ROW_BLK, D), lambda i: (i, 0)),
            pl.BlockSpec((ROW_BLK, 1), lambda i: (i, 0)),
            pl.BlockSpec((ROW_BLK, 1), lambda i: (i, 0)),
        ],
        out_shape=[
            jax.ShapeDtypeStruct((N, D), jnp.float32),
            jax.ShapeDtypeStruct((N, 1), jnp.float32),
            jax.ShapeDtypeStruct((N, 1), jnp.float32),
        ],
    )(x, W_gat, att_src.reshape(1, D), att_dst.reshape(1, D))


def _fixup_mm_body(nslope, p_ref, denom_ref, b_ref, w_ref, o_ref):
    g = p_ref[0] + p_ref[1]
    g = g * denom_ref[...] + b_ref[...]
    g = jnp.where(g >= 0, g, nslope * g)
    o_ref[...] = jnp.dot(g, w_ref[...], preferred_element_type=jnp.float32)


def _tc_fixup_mm(parts, denom_scale, b, W, nslope):
    grid = (N // ROW_BLK,)
    return pl.pallas_call(
        functools.partial(_fixup_mm_body, nslope),
        grid=grid,
        in_specs=[
            pl.BlockSpec((2, ROW_BLK, D), lambda i: (0, i, 0)),
            pl.BlockSpec((ROW_BLK, 1), lambda i: (i, 0)),
            pl.BlockSpec((1, D), lambda i: (0, 0)),
            pl.BlockSpec((D, D), lambda i: (0, 0)),
        ],
        out_specs=pl.BlockSpec((ROW_BLK, D), lambda i: (i, 0)),
        out_shape=jax.ShapeDtypeStruct((N, D), jnp.float32),
    )(parts, denom_scale, b.reshape(1, D), W)


def _fixup_body(nslope, p_ref, b_ref, o_ref):
    g = p_ref[0] + p_ref[1] + b_ref[...]
    o_ref[...] = jnp.where(g >= 0, g, nslope * g)


def _tc_fixup(parts, b, nslope):
    grid = (N // ROW_BLK,)
    return pl.pallas_call(
        functools.partial(_fixup_body, nslope),
        grid=grid,
        in_specs=[
            pl.BlockSpec((2, ROW_BLK, D), lambda i: (0, i, 0)),
            pl.BlockSpec((1, D), lambda i: (0, 0)),
        ],
        out_specs=pl.BlockSpec((ROW_BLK, D), lambda i: (i, 0)),
        out_shape=jax.ShapeDtypeStruct((N, D), jnp.float32),
    )(parts, b.reshape(1, D))


ATT_ROWS = 250


def _att_body(asrc_hbm, adst_hbm, sidx_hbm, didx_hbm, w_hbm, denom_hbm,
              sidx_v, didx_v, s_v, asv_v, adv_v, maxall_v, mtmp_v, dtmp_v,
              sbuf_v, denom_sh, maxs_sh, sem, sem2):
    cid = lax.axis_index("c")
    sid = lax.axis_index("s")
    neg = jnp.full((16,), -3e38, jnp.float32)

    @pl.when(cid == 0)
    def _pass_a():
        pltpu.sync_copy(sidx_hbm.at[sid], sidx_v)
        pltpu.sync_copy(didx_hbm.at[sid], didx_v)

        for u in range(5):
            s_v[0, pl.ds(u * 16, 16)] = jnp.zeros((16,), jnp.float32)
        for c in range(N // CHUNK):
            @pl.when(sid == c % 16)
            def _zero():
                pltpu.sync_copy(s_v.at[0], denom_sh.at[pl.ds(c * CHUNK, CHUNK)])

        asems = (sem, sem2)

        def _a_issue(r, b):
            pltpu.async_copy(asrc_hbm.at[sidx_v.at[r]], asv_v.at[b], asems[b])
            pltpu.async_copy(adst_hbm.at[didx_v.at[r]], adv_v.at[b], asems[b])

        def _a_wait(r, b):
            pltpu.make_async_copy(
                asrc_hbm.at[sidx_v.at[r]], asv_v.at[b], asems[b]).wait()
            pltpu.make_async_copy(
                adst_hbm.at[didx_v.at[r]], adv_v.at[b], asems[b]).wait()

        def _a_compute(r, b, mx):
            for u in range(5):
                sl = pl.ds(u * 16, 16)
                al = asv_v[b, sl] + adv_v[b, sl]
                al = jnp.where(al >= 0, al, 0.2 * al)
                mx = jnp.maximum(mx, al)
                s_v[r, sl] = al
            return mx

        _a_issue(0, 0)

        def pair_a(i, mx):
            r0 = 2 * i
            _a_issue(r0 + 1, 1)
            _a_wait(r0, 0)
            mx = _a_compute(r0, 0, mx)
            _a_issue(r0 + 2, 0)
            _a_wait(r0 + 1, 1)
            mx = _a_compute(r0 + 1, 1, mx)
            return mx

        mx = lax.fori_loop(0, ATT_ROWS // 2 - 1, pair_a, neg)
        r_last = ATT_ROWS - 2
        _a_issue(r_last + 1, 1)
        _a_wait(r_last, 0)
        mx = _a_compute(r_last, 0, mx)
        _a_wait(r_last + 1, 1)
        mx = _a_compute(r_last + 1, 1, mx)
        mtmp_v[pl.ds(0, 16)] = mx
        pltpu.sync_copy(mtmp_v, maxs_sh.at[sid])

    plsc.subcore_barrier()

    @pl.when(cid == 0)
    def _pass_b():
        pltpu.sync_copy(maxs_sh, maxall_v)
        mv = neg
        for i in range(16):
            mv = jnp.maximum(mv, maxall_v[i])
        for i in range(3):
            sbuf_v[pl.ds(16 * i, 16)] = neg
        sbuf_v[pl.ds(16, 16)] = mv
        for k in (8, 4, 2, 1):
            a = sbuf_v[pl.ds(16, 16)]
            b = sbuf_v[pl.ds(16 + k, 16)]
            sbuf_v[pl.ds(16, 16)] = jnp.maximum(a, b)
        for k in (1, 2, 4, 8):
            a = sbuf_v[pl.ds(16, 16)]
            b = sbuf_v[pl.ds(16 - k, 16)]
            sbuf_v[pl.ds(16, 16)] = jnp.maximum(a, b)
        gvec = sbuf_v[pl.ds(16, 16)]

        GRP = 10

        def grp_b(g, _):
            for j in range(GRP):
                r = g * GRP + j
                for u in range(5):
                    sl = pl.ds(u * 16, 16)
                    s_v[r, sl] = jnp.exp(s_v[r, sl] - gvec)
                pltpu.async_copy(s_v.at[r], denom_sh.at[didx_v.at[r]], sem,
                                 add=True)
            for j in range(GRP):
                r = g * GRP + j
                pltpu.make_async_copy(
                    s_v.at[r], denom_sh.at[didx_v.at[r]], sem).wait()
            return 0

        lax.fori_loop(0, ATT_ROWS // GRP, grp_b, 0)

    plsc.subcore_barrier()

    @pl.when(cid == 0)
    def _drain():
        pltpu.sync_copy(s_v, w_hbm.at[sid])

        @pl.when(sid < 10)
        def _dd():
            sl = pl.ds(sid * 1000, 1000)
            pltpu.sync_copy(denom_sh.at[sl], dtmp_v)
            pltpu.sync_copy(dtmp_v, denom_hbm.at[sl])


def _edge_weights_sc(a_src, a_dst, sidx3, didx3):
    mesh = plsc.VectorSubcoreMesh(core_axis_name="c", subcore_axis_name="s")
    k = functools.partial(
        pl.kernel,
        mesh=mesh,
        out_type=[
            jax.ShapeDtypeStruct((16, ATT_ROWS, CHUNK), jnp.float32),
            jax.ShapeDtypeStruct((N,), jnp.float32),
        ],
        scratch_types=[
            pltpu.VMEM((ATT_ROWS, CHUNK), jnp.int32),
            pltpu.VMEM((ATT_ROWS, CHUNK), jnp.int32),
            pltpu.VMEM((ATT_ROWS, CHUNK), jnp.float32),
            pltpu.VMEM((2, CHUNK), jnp.float32),
            pltpu.VMEM((2, CHUNK), jnp.float32),
            pltpu.VMEM((16, 16), jnp.float32),
            pltpu.VMEM((16,), jnp.float32),
            pltpu.VMEM((1000,), jnp.float32),
            pltpu.VMEM((48,), jnp.float32),
            pltpu.VMEM_SHARED((N,), jnp.float32),
            pltpu.VMEM_SHARED((16, 16), jnp.float32),
            pltpu.SemaphoreType.DMA,
            pltpu.SemaphoreType.DMA,
        ],
    )(_att_body)
    return k(a_src, a_dst, sidx3, didx3)


def _segsum_body(dense_hbm, gidx_hbm, sidx_hbm, w_hbm, out_hbm,
                 gidx_v, sidx2_v, w2_v, rows2_v, acc_sh, sem0, sem1):
    cid = lax.axis_index("c")
    sid = lax.axis_index("s")
    wid = sid * 2 + cid
    sems = (sem0, sem1)

    zvec = jnp.zeros((16,), jnp.float32)

    def zrow(i, _):
        for j in range(D // 16):
            rows2_v[0, i, pl.ds(j * 16, 16)] = zvec
        return 0

    lax.fori_loop(0, CHUNK, zrow, 0)
    for c in range(N // CHUNK):
        @pl.when(sid == c % 16)
        def _zero():
            pltpu.sync_copy(rows2_v.at[0], acc_sh.at[pl.ds(c * CHUNK, CHUNK)])

    pltpu.sync_copy(gidx_hbm.at[wid], gidx_v)
    plsc.subcore_barrier()

    def _issue(c, b):
        pltpu.async_copy(sidx_hbm.at[wid, c], sidx2_v.at[b], sems[b])
        pltpu.async_copy(w_hbm.at[wid, c], w2_v.at[b], sems[b])
        pltpu.async_copy(dense_hbm.at[gidx_v.at[c]], rows2_v.at[b], sems[b])

    def _wait(c, b):
        pltpu.make_async_copy(sidx_hbm.at[wid, c], sidx2_v.at[b], sems[b]).wait()
        pltpu.make_async_copy(w_hbm.at[wid, c], w2_v.at[b], sems[b]).wait()
        pltpu.make_async_copy(dense_hbm.at[gidx_v.at[c]], rows2_v.at[b], sems[b]).wait()

    def _scale(b):
        def group_body(u, _):
            wvec = w2_v[b, 0, pl.ds(u * 16, 16)]
            for l in range(16):
                e = u * 16 + l
                wv = wvec[l]
                for j in range(D // 16):
                    rows2_v[b, e, pl.ds(j * 16, 16)] = (
                        rows2_v[b, e, pl.ds(j * 16, 16)] * wv)
            return 0

        lax.fori_loop(0, CHUNK // 16, group_body, 0)

    def _scatter_async(b):
        pltpu.async_copy(rows2_v.at[b], acc_sh.at[sidx2_v.at[b, 0]], sems[b],
                         add=True)

    def _scatter_wait(b):
        pltpu.make_async_copy(
            rows2_v.at[b], acc_sh.at[sidx2_v.at[b, 0]], sems[b]).wait()

    _issue(0, 0)

    def pair_body(i, _):
        c0 = 2 * i
        _issue(c0 + 1, 1)
        _wait(c0, 0)
        _scale(0)
        _scatter_async(0)
        _wait(c0 + 1, 1)
        _scale(1)
        _scatter_wait(0)
        _issue(c0 + 2, 0)
        _scatter_async(1)
        _scatter_wait(1)
        return 0

    lax.fori_loop(0, (PER_TILE - 1) // 2, pair_body, 0)
    _wait(PER_TILE - 1, 0)
    _scale(0)
    _scatter_async(0)
    _scatter_wait(0)

    plsc.subcore_barrier()
    for c in range(N // CHUNK):
        @pl.when(sid == c % 16)
        def _drain():
            sl = pl.ds(c * CHUNK, CHUNK)
            pltpu.sync_copy(acc_sh.at[sl], out_hbm.at[cid, sl])


def _weighted_segsum(dense, gidx3, sidx4, w4):
    mesh = plsc.VectorSubcoreMesh(core_axis_name="c", subcore_axis_name="s")
    k = functools.partial(
        pl.kernel,
        mesh=mesh,
        out_type=jax.ShapeDtypeStruct((2, N, D), jnp.float32),
        scratch_types=[
            pltpu.VMEM((PER_TILE, CHUNK), jnp.int32),
            pltpu.VMEM((2, 1, CHUNK), jnp.int32),
            pltpu.VMEM((2, 1, CHUNK), jnp.float32),
            pltpu.VMEM((2, CHUNK, D), jnp.float32),
            pltpu.VMEM_SHARED((N, D), jnp.float32),
            pltpu.SemaphoreType.DMA,
            pltpu.SemaphoreType.DMA,
        ],
    )(_segsum_body)
    return k(dense, gidx3, sidx4, w4)


def kernel(x, edge_index, adj_values, W_gat, att_src, att_dst, b_gat, W1, b1, W2, b2):
    src = edge_index[0].astype(jnp.int32)
    dst = edge_index[1].astype(jnp.int32)

    h, a_src, a_dst = _tc_gat_mm(x, W_gat, att_src, att_dst)
    a_src = a_src.reshape(N)
    a_dst = a_dst.reshape(N)

    sidx3 = src.reshape(16, ATT_ROWS, CHUNK)
    didx3 = dst.reshape(16, ATT_ROWS, CHUNK)
    w, denom = _edge_weights_sc(a_src, a_dst, sidx3, didx3)
    inv_denom = (1.0 / (denom + 1e-16)).reshape(N, 1)

    src3 = src.reshape(TILES, PER_TILE, CHUNK)
    dst3 = dst.reshape(TILES, PER_TILE, CHUNK)
    src4 = src3.reshape(TILES, PER_TILE, 1, CHUNK)
    dst4 = dst3.reshape(TILES, PER_TILE, 1, CHUNK)
    w4 = w.reshape(TILES, PER_TILE, 1, CHUNK)
    adj4 = adj_values.reshape(TILES, PER_TILE, 1, CHUNK)

    parts = _weighted_segsum(h, src3, dst4, w4)
    support1 = _tc_fixup_mm(parts, inv_denom, b_gat, W1, 0.25)
    parts1 = _weighted_segsum(support1, dst3, src4, adj4)

    ones = jnp.ones((N, 1), jnp.float32)
    support2 = _tc_fixup_mm(parts1, ones, b1, W2, 0.25)
    parts2 = _weighted_segsum(support2, dst3, src4, adj4)

    return _tc_fixup(parts2, b2, 0.25)


# speedup vs baseline: 1.0398x; 1.0398x over previous
import functools

import jax
import jax.numpy as jnp
from jax import lax
from jax.experimental import pallas as pl
from jax.experimental.pallas import tpu as pltpu
from jax.experimental.pallas import tpu_sc as plsc

N = 10000
E = 320000
D = 128

CHUNK = 80
NROWCH = E // CHUNK
TILES = 32
PER_TILE = NROWCH // TILES

ROW_BLK = 1000


def _mm_att_body(x_ref, w_ref, asrc_ref, adst_ref, h_ref, as_ref, ad_ref):
    h = jnp.dot(x_ref[...], w_ref[...], preferred_element_type=jnp.float32)
    h_ref[...] = h
    as_ref[...] = jnp.sum(h * asrc_ref[...], axis=-1, keepdims=True)
    ad_ref[...] = jnp.sum(h * adst_ref[...], axis=-1, keepdims=True)


def _tc_gat_mm(x, W_gat, att_src, att_dst):
    grid = (N // ROW_BLK,)
    return pl.pallas_call(
        _mm_att_body,
        grid=grid,
        in_specs=[
            pl.BlockSpec((ROW_BLK, D), lambda i: (i, 0)),
            pl.BlockSpec((D, D), lambda i: (0, 0)),
            pl.BlockSpec((1, D), lambda i: (0, 0)),
            pl.BlockSpec((1, D), lambda i: (0, 0)),
        ],
        out_specs=[
            pl.BlockSpec((ROW_BLK, D), lambda i: (i, 0)),
            pl.BlockSpec((ROW_BLK, 1), lambda i: (i, 0)),
            pl.BlockSpec((ROW_BLK, 1), lambda i: (i, 0)),
        ],
        out_shape=[
            jax.ShapeDtypeStruct((N, D), jnp.float32),
            jax.ShapeDtypeStruct((N, 1), jnp.float32),
            jax.ShapeDtypeStruct((N, 1), jnp.float32),
        ],
    )(x, W_gat, att_src.reshape(1, D), att_dst.reshape(1, D))


def _fixup_mm_body(nslope, p_ref, denom_ref, b_ref, w_ref, o_ref):
    g = p_ref[0] + p_ref[1]
    g = g * denom_ref[...] + b_ref[...]
    g = jnp.where(g >= 0, g, nslope * g)
    o_ref[...] = jnp.dot(g, w_ref[...], preferred_element_type=jnp.float32)


def _tc_fixup_mm(parts, denom_scale, b, W, nslope):
    grid = (N // ROW_BLK,)
    return pl.pallas_call(
        functools.partial(_fixup_mm_body, nslope),
        grid=grid,
        in_specs=[
            pl.BlockSpec((2, ROW_BLK, D), lambda i: (0, i, 0)),
            pl.BlockSpec((ROW_BLK, 1), lambda i: (i, 0)),
            pl.BlockSpec((1, D), lambda i: (0, 0)),
            pl.BlockSpec((D, D), lambda i: (0, 0)),
        ],
        out_specs=pl.BlockSpec((ROW_BLK, D), lambda i: (i, 0)),
        out_shape=jax.ShapeDtypeStruct((N, D), jnp.float32),
    )(parts, denom_scale, b.reshape(1, D), W)


def _fixup_body(nslope, p_ref, b_ref, o_ref):
    g = p_ref[0] + p_ref[1] + b_ref[...]
    o_ref[...] = jnp.where(g >= 0, g, nslope * g)


def _tc_fixup(parts, b, nslope):
    grid = (N // ROW_BLK,)
    return pl.pallas_call(
        functools.partial(_fixup_body, nslope),
        grid=grid,
        in_specs=[
            pl.BlockSpec((2, ROW_BLK, D), lambda i: (0, i, 0)),
            pl.BlockSpec((1, D), lambda i: (0, 0)),
        ],
        out_specs=pl.BlockSpec((ROW_BLK, D), lambda i: (i, 0)),
        out_shape=jax.ShapeDtypeStruct((N, D), jnp.float32),
    )(parts, b.reshape(1, D))


ATT_ROWS = 250


def _att_body(asrc_hbm, adst_hbm, sidx_hbm, didx_hbm, w_hbm, denom_hbm,
              sidx_v, didx_v, s_v, asv_v, adv_v, maxall_v, mtmp_v, dtmp_v,
              sbuf_v, denom_sh, maxs_sh, sem, sem2):
    cid = lax.axis_index("c")
    sid = lax.axis_index("s")
    neg = jnp.full((16,), -3e38, jnp.float32)

    @pl.when(cid == 0)
    def _pass_a():
        pltpu.sync_copy(sidx_hbm.at[sid], sidx_v)
        pltpu.sync_copy(didx_hbm.at[sid], didx_v)

        for u in range(5):
            s_v[0, pl.ds(u * 16, 16)] = jnp.zeros((16,), jnp.float32)
        for c in range(N // CHUNK):
            @pl.when(sid == c % 16)
            def _zero():
                pltpu.sync_copy(s_v.at[0], denom_sh.at[pl.ds(c * CHUNK, CHUNK)])

        asems = (sem, sem2)

        def _a_issue(r, b):
            pltpu.async_copy(asrc_hbm.at[sidx_v.at[r]], asv_v.at[b], asems[b])
            pltpu.async_copy(adst_hbm.at[didx_v.at[r]], adv_v.at[b], asems[b])

        def _a_wait(r, b):
            pltpu.make_async_copy(
                asrc_hbm.at[sidx_v.at[r]], asv_v.at[b], asems[b]).wait()
            pltpu.make_async_copy(
                adst_hbm.at[didx_v.at[r]], adv_v.at[b], asems[b]).wait()

        def _a_compute(r, b, mx):
            for u in range(5):
                sl = pl.ds(u * 16, 16)
                al = asv_v[b, sl] + adv_v[b, sl]
                al = jnp.where(al >= 0, al, 0.2 * al)
                mx = jnp.maximum(mx, al)
                s_v[r, sl] = al
            return mx

        _a_issue(0, 0)

        def pair_a(i, mx):
            r0 = 2 * i
            _a_issue(r0 + 1, 1)
            _a_wait(r0, 0)
            mx = _a_compute(r0, 0, mx)
            _a_issue(r0 + 2, 0)
            _a_wait(r0 + 1, 1)
            mx = _a_compute(r0 + 1, 1, mx)
            return mx

        mx = lax.fori_loop(0, ATT_ROWS // 2 - 1, pair_a, neg)
        r_last = ATT_ROWS - 2
        _a_issue(r_last + 1, 1)
        _a_wait(r_last, 0)
        mx = _a_compute(r_last, 0, mx)
        _a_wait(r_last + 1, 1)
        mx = _a_compute(r_last + 1, 1, mx)
        mtmp_v[pl.ds(0, 16)] = mx
        pltpu.sync_copy(mtmp_v, maxs_sh.at[sid])

    plsc.subcore_barrier()

    @pl.when(cid == 0)
    def _pass_b():
        pltpu.sync_copy(maxs_sh, maxall_v)
        mv = neg
        for i in range(16):
            mv = jnp.maximum(mv, maxall_v[i])
        for i in range(3):
            sbuf_v[pl.ds(16 * i, 16)] = neg
        sbuf_v[pl.ds(16, 16)] = mv
        for k in (8, 4, 2, 1):
            a = sbuf_v[pl.ds(16, 16)]
            b = sbuf_v[pl.ds(16 + k, 16)]
            sbuf_v[pl.ds(16, 16)] = jnp.maximum(a, b)
        for k in (1, 2, 4, 8):
            a = sbuf_v[pl.ds(16, 16)]
            b = sbuf_v[pl.ds(16 - k, 16)]
            sbuf_v[pl.ds(16, 16)] = jnp.maximum(a, b)
        gvec = sbuf_v[pl.ds(16, 16)]

        GRP = 10

        def grp_b(g, _):
            for j in range(GRP):
                r = g * GRP + j
                for u in range(5):
                    sl = pl.ds(u * 16, 16)
                    s_v[r, sl] = jnp.exp(s_v[r, sl] - gvec)
                pltpu.async_copy(s_v.at[r], denom_sh.at[didx_v.at[r]], sem,
                                 add=True)
            for j in range(GRP):
                r = g * GRP + j
                pltpu.make_async_copy(
                    s_v.at[r], denom_sh.at[didx_v.at[r]], sem).wait()
            return 0

        lax.fori_loop(0, ATT_ROWS // GRP, grp_b, 0)

    plsc.subcore_barrier()

    @pl.when(cid == 0)
    def _drain():
        pltpu.sync_copy(s_v, w_hbm.at[sid])

        @pl.when(sid < 10)
        def _dd():
            sl = pl.ds(sid * 1000, 1000)
            pltpu.sync_copy(denom_sh.at[sl], dtmp_v)
            pltpu.sync_copy(dtmp_v, denom_hbm.at[sl])


def _edge_weights_sc(a_src, a_dst, sidx3, didx3):
    mesh = plsc.VectorSubcoreMesh(core_axis_name="c", subcore_axis_name="s")
    k = functools.partial(
        pl.kernel,
        mesh=mesh,
        out_type=[
            jax.ShapeDtypeStruct((16, ATT_ROWS, CHUNK), jnp.float32),
            jax.ShapeDtypeStruct((N,), jnp.float32),
        ],
        scratch_types=[
            pltpu.VMEM((ATT_ROWS, CHUNK), jnp.int32),
            pltpu.VMEM((ATT_ROWS, CHUNK), jnp.int32),
            pltpu.VMEM((ATT_ROWS, CHUNK), jnp.float32),
            pltpu.VMEM((2, CHUNK), jnp.float32),
            pltpu.VMEM((2, CHUNK), jnp.float32),
            pltpu.VMEM((16, 16), jnp.float32),
            pltpu.VMEM((16,), jnp.float32),
            pltpu.VMEM((1000,), jnp.float32),
            pltpu.VMEM((48,), jnp.float32),
            pltpu.VMEM_SHARED((N,), jnp.float32),
            pltpu.VMEM_SHARED((16, 16), jnp.float32),
            pltpu.SemaphoreType.DMA,
            pltpu.SemaphoreType.DMA,
        ],
    )(_att_body)
    return k(a_src, a_dst, sidx3, didx3)


def _segsum_body(dense_hbm, gidx_hbm, sidx_hbm, w_hbm, out_hbm,
                 gidx_v, sidx2_v, w2_v, rows2_v, acc_sh, sem0, sem1):
    cid = lax.axis_index("c")
    sid = lax.axis_index("s")
    wid = sid * 2 + cid
    sems = (sem0, sem1)

    zvec = jnp.zeros((16,), jnp.float32)

    def zrow(i, _):
        for j in range(D // 16):
            rows2_v[0, i, pl.ds(j * 16, 16)] = zvec
        return 0

    lax.fori_loop(0, CHUNK, zrow, 0)
    for c in range(N // CHUNK):
        @pl.when(sid == c % 16)
        def _zero():
            pltpu.sync_copy(rows2_v.at[0], acc_sh.at[pl.ds(c * CHUNK, CHUNK)])

    pltpu.sync_copy(gidx_hbm.at[wid], gidx_v)
    plsc.subcore_barrier()

    def _issue(c, b):
        pltpu.async_copy(sidx_hbm.at[wid, c], sidx2_v.at[b], sems[b])
        pltpu.async_copy(w_hbm.at[wid, c], w2_v.at[b], sems[b])
        pltpu.async_copy(dense_hbm.at[gidx_v.at[c]], rows2_v.at[b], sems[b])

    def _wait(c, b):
        pltpu.make_async_copy(sidx_hbm.at[wid, c], sidx2_v.at[b], sems[b]).wait()
        pltpu.make_async_copy(w_hbm.at[wid, c], w2_v.at[b], sems[b]).wait()
        pltpu.make_async_copy(dense_hbm.at[gidx_v.at[c]], rows2_v.at[b], sems[b]).wait()

    def _scale(b):
        def group_body(u, _):
            wvec = w2_v[b, 0, pl.ds(u * 16, 16)]
            for l in range(16):
                e = u * 16 + l
                wv = wvec[l]
                for j in range(D // 16):
                    rows2_v[b, e, pl.ds(j * 16, 16)] = (
                        rows2_v[b, e, pl.ds(j * 16, 16)] * wv)
            return 0

        lax.fori_loop(0, CHUNK // 16, group_body, 0)
        pltpu.sync_copy(rows2_v.at[b], acc_sh.at[sidx2_v.at[b, 0]], add=True)

    _issue(0, 0)

    def pair_body(i, _):
        c0 = 2 * i
        _issue(c0 + 1, 1)
        _wait(c0, 0)
        _scale(0)
        _issue(c0 + 2, 0)
        _wait(c0 + 1, 1)
        _scale(1)
        return 0

    lax.fori_loop(0, (PER_TILE - 1) // 2, pair_body, 0)
    _wait(PER_TILE - 1, 0)
    _scale(0)

    plsc.subcore_barrier()
    for c in range(N // CHUNK):
        @pl.when(sid == c % 16)
        def _drain():
            sl = pl.ds(c * CHUNK, CHUNK)
            pltpu.sync_copy(acc_sh.at[sl], out_hbm.at[cid, sl])


def _weighted_segsum(dense, gidx3, sidx4, w4):
    mesh = plsc.VectorSubcoreMesh(core_axis_name="c", subcore_axis_name="s")
    k = functools.partial(
        pl.kernel,
        mesh=mesh,
        out_type=jax.ShapeDtypeStruct((2, N, D), jnp.float32),
        scratch_types=[
            pltpu.VMEM((PER_TILE, CHUNK), jnp.int32),
            pltpu.VMEM((2, 1, CHUNK), jnp.int32),
            pltpu.VMEM((2, 1, CHUNK), jnp.float32),
            pltpu.VMEM((2, CHUNK, D), jnp.float32),
            pltpu.VMEM_SHARED((N, D), jnp.float32),
            pltpu.SemaphoreType.DMA,
            pltpu.SemaphoreType.DMA,
        ],
    )(_segsum_body)
    return k(dense, gidx3, sidx4, w4)


def kernel(x, edge_index, adj_values, W_gat, att_src, att_dst, b_gat, W1, b1, W2, b2):
    src = edge_index[0].astype(jnp.int32)
    dst = edge_index[1].astype(jnp.int32)

    h, a_src, a_dst = _tc_gat_mm(x, W_gat, att_src, att_dst)
    a_src = a_src.reshape(N)
    a_dst = a_dst.reshape(N)

    sidx3 = src.reshape(16, ATT_ROWS, CHUNK)
    didx3 = dst.reshape(16, ATT_ROWS, CHUNK)
    w, denom = _edge_weights_sc(a_src, a_dst, sidx3, didx3)
    inv_denom = (1.0 / (denom + 1e-16)).reshape(N, 1)

    src3 = src.reshape(TILES, PER_TILE, CHUNK)
    dst3 = dst.reshape(TILES, PER_TILE, CHUNK)
    src4 = src3.reshape(TILES, PER_TILE, 1, CHUNK)
    dst4 = dst3.reshape(TILES, PER_TILE, 1, CHUNK)
    w4 = w.reshape(TILES, PER_TILE, 1, CHUNK)
    adj4 = adj_values.reshape(TILES, PER_TILE, 1, CHUNK)

    parts = _weighted_segsum(h, src3, dst4, w4)
    support1 = _tc_fixup_mm(parts, inv_denom, b_gat, W1, 0.25)
    parts1 = _weighted_segsum(support1, dst3, src4, adj4)

    ones = jnp.ones((N, 1), jnp.float32)
    support2 = _tc_fixup_mm(parts1, ones, b1, W2, 0.25)
    parts2 = _weighted_segsum(support2, dst3, src4, adj4)

    return _tc_fixup(parts2, b2, 0.25)
